# baseline (speedup 1.0000x reference)
import functools

import jax
import jax.numpy as jnp
from jax import lax
from jax.experimental import pallas as pl
from jax.experimental.pallas import tpu as pltpu
from jax.experimental.pallas import tpu_sc as plsc

N = 10000
H = 128
G = 128
NC = 2
NS = 16
NW = NC * NS
EC = 128
N_ACC = 10112
RPT = N_ACC // NS
BLK = 2000


def _sc_agg(h, src_w, dst_w, zrows):
    C = src_w.shape[1]
    NST = 2
    CH = C // NST
    NB = 2

    @functools.partial(
        pl.kernel,
        out_type=jax.ShapeDtypeStruct((NC, N_ACC, H), jnp.float32),
        mesh=plsc.VectorSubcoreMesh(core_axis_name="c", subcore_axis_name="s"),
        scratch_types=[
            pltpu.VMEM_SHARED((N_ACC, H), jnp.float32),
            pltpu.VMEM((CH, EC), jnp.int32),
            pltpu.VMEM((CH, EC), jnp.int32),
            [pltpu.VMEM((EC, H), jnp.float32) for _ in range(NB)],
            [pltpu.SemaphoreType.DMA for _ in range(NB)],
            pltpu.SemaphoreType.DMA,
            pltpu.SemaphoreType.DMA,
        ],
    )
    def k(h_hbm, src_hbm, dst_hbm, z_hbm, out_hbm,
          acc, src_v, dst_v, bufs, gsems, zsem, isem):
        cid = lax.axis_index("c")
        sid = lax.axis_index("s")
        wid = cid * NS + sid

        def gather(c, j):
            pltpu.async_copy(h_hbm.at[src_v.at[c]], bufs[j], gsems[j])

        def gather_wait(c, j):
            pltpu.make_async_copy(h_hbm.at[src_v.at[c]], bufs[j], gsems[j]).wait()

        if True:
            zcp = pltpu.async_copy(z_hbm, acc.at[pl.ds(sid * RPT, RPT)], zsem)
            pltpu.async_copy(src_hbm.at[wid].at[pl.ds(0, CH)], src_v, isem)
            icp = pltpu.async_copy(dst_hbm.at[wid].at[pl.ds(0, CH)], dst_v, isem)
            icp.wait()
            pltpu.make_async_copy(src_hbm.at[wid].at[pl.ds(0, CH)],
                                  src_v, isem).wait()
            gather(0, 0)
            gather(1, 1)
            zcp.wait()
            plsc.subcore_barrier()

        for stage in range(NST):
            base = stage * CH
            if stage > 0:
                if True:
                    pltpu.sync_copy(src_hbm.at[wid].at[pl.ds(base, CH)], src_v)
                    pltpu.sync_copy(dst_hbm.at[wid].at[pl.ds(base, CH)], dst_v)

            if True:
                if stage > 0:
                    gather(0, 0)
                    gather(1, 1)

                @pl.loop(0, CH // NB)
                def _(r):
                    for j in range(NB):
                        c = r * NB + j
                        gather_wait(c, j)
                        pltpu.sync_copy(bufs[j], acc.at[dst_v.at[c]], add=True)

                        @pl.when(c + NB < CH)
                        def _():
                            gather(c + NB, j)

        plsc.subcore_barrier()
        if True:
            r = sid * RPT
            pltpu.sync_copy(acc.at[pl.ds(r, RPT)],
                            out_hbm.at[cid].at[pl.ds(r, RPT)])

    return k(h, src_w, dst_w, zrows)


def _tc_pad_edges(srcr, dstr, C):
    ER = srcr.shape[0]
    PR = NW * C * EC // 128

    def body(s_ref, d_ref, os_ref, od_ref):
        os_ref[0:ER, :] = s_ref[...]
        od_ref[0:ER, :] = d_ref[...]
        rows = lax.broadcasted_iota(jnp.int32, (PR - ER, 128), 0)
        lanes = lax.broadcasted_iota(jnp.int32, (PR - ER, 128), 1)
        p = rows * 128 + lanes
        os_ref[ER:PR, :] = p % N
        od_ref[ER:PR, :] = N + p % (N_ACC - N)

    return pl.pallas_call(
        body,
        out_shape=[jax.ShapeDtypeStruct((PR, 128), jnp.int32),
                   jax.ShapeDtypeStruct((PR, 128), jnp.int32)],
    )(srcr, dstr)


def _eps_spec():
    return pl.BlockSpec(memory_space=pltpu.SMEM)


def _wspec():
    return pl.BlockSpec((H, H), lambda i: (0, 0))


def _bspec():
    return pl.BlockSpec((1, H), lambda i: (0, 0))


def _rowspec():
    return pl.BlockSpec((BLK, H), lambda i: (i, 0))


def _pspec():
    return pl.BlockSpec((2, BLK, H), lambda i: (0, i, 0))


def _sspec():
    return pl.BlockSpec((8, H), lambda i: (0, 0))


def _tc_conv1(eps, x, p, W1, b1, W2, b2):
    def body(eps_ref, x_ref, p_ref, w1_ref, b1_ref, w2_ref, b2_ref, o_ref):
        a = (1.0 + eps_ref[0, 0]) * x_ref[...] + p_ref[0] + p_ref[1]
        t = jnp.dot(a, w1_ref[...], preferred_element_type=jnp.float32)
        t = jnp.maximum(t + b1_ref[...], 0.0)
        t = jnp.dot(t, w2_ref[...], preferred_element_type=jnp.float32)
        o_ref[...] = jnp.maximum(t + b2_ref[...], 0.0)

    return pl.pallas_call(
        body,
        grid=(N // BLK,),
        in_specs=[_eps_spec(), _rowspec(), _pspec(),
                  _wspec(), _bspec(), _wspec(), _bspec()],
        out_specs=_rowspec(),
        out_shape=jax.ShapeDtypeStruct((N, H), jnp.float32),
    )(eps, x, p, W1, b1, W2, b2)


def _tc_conv_bn(eps, hin, p, W1, b1, g1, be1, W2, b2, g2, be2):
    nb = N // BLK

    def stats_update(s_ref, z, first):
        @pl.when(first)
        def _():
            s_ref[...] = jnp.zeros_like(s_ref)

        su = jnp.sum(z, axis=0, keepdims=True)
        sq = jnp.sum(z * z, axis=0, keepdims=True)
        s_ref[...] += jnp.concatenate(
            [su, sq, jnp.zeros((6, H), jnp.float32)], axis=0)

    def bn(z, s_ref, g_ref, be_ref):
        m = s_ref[0:1, :] * (1.0 / N)
        v = s_ref[1:2, :] * (1.0 / N) - m * m
        inv = lax.rsqrt(v + 1e-5)
        return (z - m) * inv * g_ref[...] + be_ref[...]

    def body(eps_ref, h_ref, p_ref, w1_ref, b1_ref, g1_ref, be1_ref,
             w2_ref, b2_ref, g2_ref, be2_ref, o_ref, z1s, z2s, s1, s2):
        i = pl.program_id(0)

        @pl.when(i < nb)
        def _():
            a = (1.0 + eps_ref[0, 0]) * h_ref[...] + p_ref[0] + p_ref[1]
            z = jnp.dot(a, w1_ref[...], preferred_element_type=jnp.float32)
            z = z + b1_ref[...]
            z1s[pl.ds(pl.multiple_of(i * BLK, BLK), BLK), :] = z
            stats_update(s1, z, i == 0)

        @pl.when(jnp.logical_and(i >= nb, i < 2 * nb))
        def _():
            ib = pl.multiple_of((i - nb) * BLK, BLK)
            z = z1s[pl.ds(ib, BLK), :]
            hmid = jnp.maximum(bn(z, s1, g1_ref, be1_ref), 0.0)
            z2 = jnp.dot(hmid, w2_ref[...], preferred_element_type=jnp.float32)
            z2 = z2 + b2_ref[...]
            z2s[pl.ds(ib, BLK), :] = z2
            stats_update(s2, z2, i == nb)

        @pl.when(i >= 2 * nb)
        def _():
            ib = pl.multiple_of((i - 2 * nb) * BLK, BLK)
            z2 = z2s[pl.ds(ib, BLK), :]
            o_ref[...] = jnp.maximum(bn(z2, s2, g2_ref, be2_ref), 0.0)

    def clamp_row(i):
        return (jnp.minimum(i, nb - 1), 0)

    def clamp_p(i):
        return (0, jnp.minimum(i, nb - 1), 0)

    return pl.pallas_call(
        body,
        grid=(3 * nb,),
        in_specs=[_eps_spec(),
                  pl.BlockSpec((BLK, H), clamp_row),
                  pl.BlockSpec((2, BLK, H), clamp_p),
                  _wspec(), _bspec(), _bspec(), _bspec(),
                  _wspec(), _bspec(), _bspec(), _bspec()],
        out_specs=pl.BlockSpec((BLK, H),
                               lambda i: (jnp.maximum(i - 2 * nb, 0), 0)),
        out_shape=jax.ShapeDtypeStruct((N, H), jnp.float32),
        scratch_shapes=[pltpu.VMEM((N, H), jnp.float32),
                        pltpu.VMEM((N, H), jnp.float32),
                        pltpu.VMEM((8, H), jnp.float32),
                        pltpu.VMEM((8, H), jnp.float32)],
    )(eps, hin, p, W1, b1, g1, be1, W2, b2, g2, be2)


def _tc_lin_stats(eps, hin, p, W, b):
    nb = N // BLK

    def body(eps_ref, h_ref, p_ref, w_ref, b_ref, z_ref, s_ref):
        i = pl.program_id(0)
        a = (1.0 + eps_ref[0, 0]) * h_ref[...] + p_ref[0] + p_ref[1]
        z = jnp.dot(a, w_ref[...], preferred_element_type=jnp.float32) + b_ref[...]
        z_ref[...] = z

        @pl.when(i == 0)
        def _():
            s_ref[...] = jnp.zeros_like(s_ref)

        su = jnp.sum(z, axis=0, keepdims=True)
        sq = jnp.sum(z * z, axis=0, keepdims=True)
        s_ref[...] += jnp.concatenate(
            [su, sq, jnp.zeros((6, H), jnp.float32)], axis=0)

    return pl.pallas_call(
        body,
        grid=(nb,),
        in_specs=[_eps_spec(), _rowspec(), _pspec(), _wspec(), _bspec()],
        out_specs=[_rowspec(), _sspec()],
        out_shape=[jax.ShapeDtypeStruct((N, H), jnp.float32),
                   jax.ShapeDtypeStruct((8, H), jnp.float32)],
    )(eps, hin, p, W, b)


def _tc_bn_lin_stats(z, s, g, be, W, b):
    nb = N // BLK

    def body(z_ref, s_ref, g_ref, be_ref, w_ref, b_ref, z2_ref, s2_ref):
        i = pl.program_id(0)
        m = s_ref[0:1, :] * (1.0 / N)
        v = s_ref[1:2, :] * (1.0 / N) - m * m
        inv = lax.rsqrt(v + 1e-5)
        hmid = jnp.maximum((z_ref[...] - m) * inv * g_ref[...] + be_ref[...], 0.0)
        z2 = jnp.dot(hmid, w_ref[...], preferred_element_type=jnp.float32) + b_ref[...]
        z2_ref[...] = z2

        @pl.when(i == 0)
        def _():
            s2_ref[...] = jnp.zeros_like(s2_ref)

        su = jnp.sum(z2, axis=0, keepdims=True)
        sq = jnp.sum(z2 * z2, axis=0, keepdims=True)
        s2_ref[...] += jnp.concatenate(
            [su, sq, jnp.zeros((6, H), jnp.float32)], axis=0)

    return pl.pallas_call(
        body,
        grid=(nb,),
        in_specs=[_rowspec(), _sspec(), _bspec(), _bspec(), _wspec(), _bspec()],
        out_specs=[_rowspec(), _sspec()],
        out_shape=[jax.ShapeDtypeStruct((N, H), jnp.float32),
                   jax.ShapeDtypeStruct((8, H), jnp.float32)],
    )(z, s, g, be, W, b)


def _tc_bn_relu(z, s, g, be):
    def body(z_ref, s_ref, g_ref, be_ref, o_ref):
        m = s_ref[0:1, :] * (1.0 / N)
        v = s_ref[1:2, :] * (1.0 / N) - m * m
        inv = lax.rsqrt(v + 1e-5)
        o_ref[...] = jnp.maximum(
            (z_ref[...] - m) * inv * g_ref[...] + be_ref[...], 0.0)

    return pl.pallas_call(
        body,
        grid=(N // BLK,),
        in_specs=[_rowspec(), _sspec(), _bspec(), _bspec()],
        out_specs=_rowspec(),
        out_shape=jax.ShapeDtypeStruct((N, H), jnp.float32),
    )(z, s, g, be)


def _tc_head(h1, h2, h3, batch2, Wa, Wb, Wc, b, pW, pb):
    nb = N // BLK

    def body(h1_ref, h2_ref, h3_ref, bt_ref, wa_ref, wb_ref, wc_ref, b_ref,
             pw_ref, pb_ref, out_ref, s_ref, c_ref):
        i = pl.program_id(0)
        hl = (jnp.dot(h1_ref[...], wa_ref[...], preferred_element_type=jnp.float32)
              + jnp.dot(h2_ref[...], wb_ref[...], preferred_element_type=jnp.float32)
              + jnp.dot(h3_ref[...], wc_ref[...], preferred_element_type=jnp.float32)
              + b_ref[...])
        hl = jnp.maximum(hl, 0.0)
        gids = lax.broadcasted_iota(jnp.int32, (G, BLK), 0)
        oh = (bt_ref[0] == gids).astype(jnp.float32)
        ps = jnp.dot(oh, hl, preferred_element_type=jnp.float32)
        cs = jnp.dot(oh, jnp.ones((BLK, H), jnp.float32),
                     preferred_element_type=jnp.float32)

        @pl.when(i == 0)
        def _():
            s_ref[...] = jnp.zeros_like(s_ref)
            c_ref[...] = jnp.zeros_like(c_ref)

        s_ref[...] += ps
        c_ref[...] += cs

        @pl.when(i == nb - 1)
        def _():
            pooled = s_ref[...] / jnp.maximum(c_ref[...], 1.0)
            o = jnp.dot(pooled, pw_ref[...], preferred_element_type=jnp.float32)
            out_ref[...] = jnp.maximum(o + pb_ref[...], 0.0)

    out, _, _ = pl.pallas_call(
        body,
        grid=(nb,),
        in_specs=[_rowspec(), _rowspec(), _rowspec(),
                  pl.BlockSpec((1, 1, BLK), lambda i: (i, 0, 0)),
                  _wspec(), _wspec(), _wspec(), _bspec(), _wspec(), _bspec()],
        out_specs=[pl.BlockSpec((G, H), lambda i: (0, 0)),
                   pl.BlockSpec((G, H), lambda i: (0, 0)),
                   pl.BlockSpec((G, H), lambda i: (0, 0))],
        out_shape=[jax.ShapeDtypeStruct((G, H), jnp.float32),
                   jax.ShapeDtypeStruct((G, H), jnp.float32),
                   jax.ShapeDtypeStruct((G, H), jnp.float32)],
    )(h1, h2, h3, batch2, Wa, Wb, Wc, b, pW, pb)
    return out


def kernel(x, edge_index, batch, params):
    p = params
    src = edge_index[0]
    dst = edge_index[1]
    E = src.shape[0]
    C = -(-E // (NW * EC))
    C = -(-C // 16) * 16
    src_p, dst_p = _tc_pad_edges(src.reshape(E // 128, 128),
                                 dst.reshape(E // 128, 128), C)
    src_w = src_p.reshape(NW, C, EC)
    dst_w = dst_p.reshape(NW, C, EC)
    zrows = jnp.zeros((RPT, H), jnp.float32)
    batch2 = batch.reshape(N // BLK, 1, BLK)

    def r2(a):
        return a.reshape(1, H)

    eps1 = p['eps1'].reshape(1, 1)
    eps2 = p['eps2'].reshape(1, 1)
    eps3 = p['eps3'].reshape(1, 1)

    pr = _sc_agg(x, src_w, dst_w, zrows)
    h1 = _tc_conv1(eps1, x, pr, p['c1_W1'], r2(p['c1_b1']),
                   p['c1_W2'], r2(p['c1_b2']))

    pr = _sc_agg(h1, src_w, dst_w, zrows)
    h2 = _tc_conv_bn(eps2, h1, pr, p['c2_W1'], r2(p['c2_b1']),
                     r2(p['c2_g1']), r2(p['c2_be1']),
                     p['c2_W2'], r2(p['c2_b2']),
                     r2(p['c2_g2']), r2(p['c2_be2']))

    pr = _sc_agg(h2, src_w, dst_w, zrows)
    h3 = _tc_conv_bn(eps3, h2, pr, p['c3_W1'], r2(p['c3_b1']),
                     r2(p['c3_g1']), r2(p['c3_be1']),
                     p['c3_W2'], r2(p['c3_b2']),
                     r2(p['c3_g2']), r2(p['c3_be2']))

    W = p['lin1_W']
    out = _tc_head(h1, h2, h3, batch2,
                   W[0:H], W[H:2 * H], W[2 * H:3 * H], r2(p['lin1_b']),
                   p['pred_W'], r2(p['pred_b']))
    return out


# speedup vs baseline: 1.0037x; 1.0037x over previous
import functools

import jax
import jax.numpy as jnp
from jax import lax
from jax.experimental import pallas as pl
from jax.experimental.pallas import tpu as pltpu
from jax.experimental.pallas import tpu_sc as plsc

N = 10000
H = 128
G = 128
NC = 2
NS = 16
NW = NC * NS
EC = 128
N_ACC = 10112
RPT = N_ACC // NS
BLK = 2000


def _sc_agg(h, src_w, dst_w, zrows):
    C = src_w.shape[1]
    NST = 2
    CH = C // NST
    NB = 2

    @functools.partial(
        pl.kernel,
        out_type=jax.ShapeDtypeStruct((NC, N_ACC, H), jnp.float32),
        mesh=plsc.VectorSubcoreMesh(core_axis_name="c", subcore_axis_name="s"),
        scratch_types=[
            pltpu.VMEM_SHARED((N_ACC, H), jnp.float32),
            pltpu.VMEM((CH, EC), jnp.int32),
            pltpu.VMEM((CH, EC), jnp.int32),
            [pltpu.VMEM((EC, H), jnp.float32) for _ in range(NB)],
            [pltpu.SemaphoreType.DMA for _ in range(NB)],
            pltpu.SemaphoreType.DMA,
            pltpu.SemaphoreType.DMA,
        ],
    )
    def k(h_hbm, src_hbm, dst_hbm, z_hbm, out_hbm,
          acc, src_v, dst_v, bufs, gsems, zsem, isem):
        cid = lax.axis_index("c")
        sid = lax.axis_index("s")
        wid = cid * NS + sid

        def gather(c, j):
            pltpu.async_copy(h_hbm.at[src_v.at[c]], bufs[j], gsems[j])

        def gather_wait(c, j):
            pltpu.make_async_copy(h_hbm.at[src_v.at[c]], bufs[j], gsems[j]).wait()

        with jax.named_scope("zinit"):
            zcp = pltpu.async_copy(z_hbm, acc.at[pl.ds(sid * RPT, RPT)], zsem)
            pltpu.async_copy(src_hbm.at[wid].at[pl.ds(0, CH)], src_v, isem)
            icp = pltpu.async_copy(dst_hbm.at[wid].at[pl.ds(0, CH)], dst_v, isem)
            icp.wait()
            pltpu.make_async_copy(src_hbm.at[wid].at[pl.ds(0, CH)],
                                  src_v, isem).wait()
            gather(0, 0)
            gather(1, 1)
            zcp.wait()
            plsc.subcore_barrier()

        for stage in range(NST):
            base = stage * CH
            if stage > 0:
                with jax.named_scope("idx"):
                    pltpu.sync_copy(src_hbm.at[wid].at[pl.ds(base, CH)], src_v)
                    pltpu.sync_copy(dst_hbm.at[wid].at[pl.ds(base, CH)], dst_v)

            with jax.named_scope("gsloop"):
                if stage > 0:
                    gather(0, 0)
                    gather(1, 1)

                @pl.loop(0, CH // NB)
                def _(r):
                    for j in range(NB):
                        c = r * NB + j
                        gather_wait(c, j)
                        pltpu.sync_copy(bufs[j], acc.at[dst_v.at[c]], add=True)

                        @pl.when(c + NB < CH)
                        def _():
                            gather(c + NB, j)

        plsc.subcore_barrier()
        with jax.named_scope("wout"):
            r = sid * RPT
            pltpu.sync_copy(acc.at[pl.ds(r, RPT)],
                            out_hbm.at[cid].at[pl.ds(r, RPT)])

    return k(h, src_w, dst_w, zrows)


def _tc_pad_edges(srcr, dstr, C):
    ER = srcr.shape[0]
    PR = NW * C * EC // 128

    def body(s_ref, d_ref, os_ref, od_ref):
        os_ref[0:ER, :] = s_ref[...]
        od_ref[0:ER, :] = d_ref[...]
        rows = lax.broadcasted_iota(jnp.int32, (PR - ER, 128), 0)
        lanes = lax.broadcasted_iota(jnp.int32, (PR - ER, 128), 1)
        p = rows * 128 + lanes
        os_ref[ER:PR, :] = p % N
        od_ref[ER:PR, :] = N + p % (N_ACC - N)

    return pl.pallas_call(
        body,
        out_shape=[jax.ShapeDtypeStruct((PR, 128), jnp.int32),
                   jax.ShapeDtypeStruct((PR, 128), jnp.int32)],
    )(srcr, dstr)


def _eps_spec():
    return pl.BlockSpec(memory_space=pltpu.SMEM)


def _wspec():
    return pl.BlockSpec((H, H), lambda i: (0, 0))


def _bspec():
    return pl.BlockSpec((1, H), lambda i: (0, 0))


def _rowspec():
    return pl.BlockSpec((BLK, H), lambda i: (i, 0))


def _pspec():
    return pl.BlockSpec((2, BLK, H), lambda i: (0, i, 0))


def _sspec():
    return pl.BlockSpec((8, H), lambda i: (0, 0))


def _tc_conv1(eps, x, p, W1, b1, W2, b2):
    def body(eps_ref, x_ref, p_ref, w1_ref, b1_ref, w2_ref, b2_ref, o_ref):
        a = (1.0 + eps_ref[0, 0]) * x_ref[...] + p_ref[0] + p_ref[1]
        t = jnp.dot(a, w1_ref[...], preferred_element_type=jnp.float32)
        t = jnp.maximum(t + b1_ref[...], 0.0)
        t = jnp.dot(t, w2_ref[...], preferred_element_type=jnp.float32)
        o_ref[...] = jnp.maximum(t + b2_ref[...], 0.0)

    return pl.pallas_call(
        body,
        grid=(N // BLK,),
        in_specs=[_eps_spec(), _rowspec(), _pspec(),
                  _wspec(), _bspec(), _wspec(), _bspec()],
        out_specs=_rowspec(),
        out_shape=jax.ShapeDtypeStruct((N, H), jnp.float32),
    )(eps, x, p, W1, b1, W2, b2)


def _tc_conv_bn(eps, hin, p, W1, b1, g1, be1, W2, b2, g2, be2):
    nb = N // BLK

    def stats_update(s_ref, z, first):
        @pl.when(first)
        def _():
            s_ref[...] = jnp.zeros_like(s_ref)

        su = jnp.sum(z, axis=0, keepdims=True)
        sq = jnp.sum(z * z, axis=0, keepdims=True)
        s_ref[...] += jnp.concatenate(
            [su, sq, jnp.zeros((6, H), jnp.float32)], axis=0)

    def bn(z, s_ref, g_ref, be_ref):
        m = s_ref[0:1, :] * (1.0 / N)
        v = s_ref[1:2, :] * (1.0 / N) - m * m
        inv = lax.rsqrt(v + 1e-5)
        return (z - m) * inv * g_ref[...] + be_ref[...]

    def body(eps_ref, h_ref, p_ref, w1_ref, b1_ref, g1_ref, be1_ref,
             w2_ref, b2_ref, g2_ref, be2_ref, o_ref, z1s, z2s, s1, s2):
        i = pl.program_id(0)

        @pl.when(i < nb)
        def _():
            a = (1.0 + eps_ref[0, 0]) * h_ref[...] + p_ref[0] + p_ref[1]
            z = jnp.dot(a, w1_ref[...], preferred_element_type=jnp.float32)
            z = z + b1_ref[...]
            z1s[pl.ds(pl.multiple_of(i * BLK, BLK), BLK), :] = z
            stats_update(s1, z, i == 0)

        @pl.when(jnp.logical_and(i >= nb, i < 2 * nb))
        def _():
            ib = pl.multiple_of((i - nb) * BLK, BLK)
            z = z1s[pl.ds(ib, BLK), :]
            hmid = jnp.maximum(bn(z, s1, g1_ref, be1_ref), 0.0)
            z2 = jnp.dot(hmid, w2_ref[...], preferred_element_type=jnp.float32)
            z2 = z2 + b2_ref[...]
            z2s[pl.ds(ib, BLK), :] = z2
            stats_update(s2, z2, i == nb)

        @pl.when(i >= 2 * nb)
        def _():
            ib = pl.multiple_of((i - 2 * nb) * BLK, BLK)
            z2 = z2s[pl.ds(ib, BLK), :]
            o_ref[...] = jnp.maximum(bn(z2, s2, g2_ref, be2_ref), 0.0)

    def clamp_row(i):
        return (jnp.minimum(i, nb - 1), 0)

    def clamp_p(i):
        return (0, jnp.minimum(i, nb - 1), 0)

    return pl.pallas_call(
        body,
        grid=(3 * nb,),
        in_specs=[_eps_spec(),
                  pl.BlockSpec((BLK, H), clamp_row),
                  pl.BlockSpec((2, BLK, H), clamp_p),
                  _wspec(), _bspec(), _bspec(), _bspec(),
                  _wspec(), _bspec(), _bspec(), _bspec()],
        out_specs=pl.BlockSpec((BLK, H),
                               lambda i: (jnp.maximum(i - 2 * nb, 0), 0)),
        out_shape=jax.ShapeDtypeStruct((N, H), jnp.float32),
        scratch_shapes=[pltpu.VMEM((N, H), jnp.float32),
                        pltpu.VMEM((N, H), jnp.float32),
                        pltpu.VMEM((8, H), jnp.float32),
                        pltpu.VMEM((8, H), jnp.float32)],
    )(eps, hin, p, W1, b1, g1, be1, W2, b2, g2, be2)


def _tc_lin_stats(eps, hin, p, W, b):
    nb = N // BLK

    def body(eps_ref, h_ref, p_ref, w_ref, b_ref, z_ref, s_ref):
        i = pl.program_id(0)
        a = (1.0 + eps_ref[0, 0]) * h_ref[...] + p_ref[0] + p_ref[1]
        z = jnp.dot(a, w_ref[...], preferred_element_type=jnp.float32) + b_ref[...]
        z_ref[...] = z

        @pl.when(i == 0)
        def _():
            s_ref[...] = jnp.zeros_like(s_ref)

        su = jnp.sum(z, axis=0, keepdims=True)
        sq = jnp.sum(z * z, axis=0, keepdims=True)
        s_ref[...] += jnp.concatenate(
            [su, sq, jnp.zeros((6, H), jnp.float32)], axis=0)

    return pl.pallas_call(
        body,
        grid=(nb,),
        in_specs=[_eps_spec(), _rowspec(), _pspec(), _wspec(), _bspec()],
        out_specs=[_rowspec(), _sspec()],
        out_shape=[jax.ShapeDtypeStruct((N, H), jnp.float32),
                   jax.ShapeDtypeStruct((8, H), jnp.float32)],
    )(eps, hin, p, W, b)


def _tc_bn_lin_stats(z, s, g, be, W, b):
    nb = N // BLK

    def body(z_ref, s_ref, g_ref, be_ref, w_ref, b_ref, z2_ref, s2_ref):
        i = pl.program_id(0)
        m = s_ref[0:1, :] * (1.0 / N)
        v = s_ref[1:2, :] * (1.0 / N) - m * m
        inv = lax.rsqrt(v + 1e-5)
        hmid = jnp.maximum((z_ref[...] - m) * inv * g_ref[...] + be_ref[...], 0.0)
        z2 = jnp.dot(hmid, w_ref[...], preferred_element_type=jnp.float32) + b_ref[...]
        z2_ref[...] = z2

        @pl.when(i == 0)
        def _():
            s2_ref[...] = jnp.zeros_like(s2_ref)

        su = jnp.sum(z2, axis=0, keepdims=True)
        sq = jnp.sum(z2 * z2, axis=0, keepdims=True)
        s2_ref[...] += jnp.concatenate(
            [su, sq, jnp.zeros((6, H), jnp.float32)], axis=0)

    return pl.pallas_call(
        body,
        grid=(nb,),
        in_specs=[_rowspec(), _sspec(), _bspec(), _bspec(), _wspec(), _bspec()],
        out_specs=[_rowspec(), _sspec()],
        out_shape=[jax.ShapeDtypeStruct((N, H), jnp.float32),
                   jax.ShapeDtypeStruct((8, H), jnp.float32)],
    )(z, s, g, be, W, b)


def _tc_bn_relu(z, s, g, be):
    def body(z_ref, s_ref, g_ref, be_ref, o_ref):
        m = s_ref[0:1, :] * (1.0 / N)
        v = s_ref[1:2, :] * (1.0 / N) - m * m
        inv = lax.rsqrt(v + 1e-5)
        o_ref[...] = jnp.maximum(
            (z_ref[...] - m) * inv * g_ref[...] + be_ref[...], 0.0)

    return pl.pallas_call(
        body,
        grid=(N // BLK,),
        in_specs=[_rowspec(), _sspec(), _bspec(), _bspec()],
        out_specs=_rowspec(),
        out_shape=jax.ShapeDtypeStruct((N, H), jnp.float32),
    )(z, s, g, be)


def _tc_head(h1, h2, h3, batch2, Wa, Wb, Wc, b, pW, pb):
    nb = N // BLK

    def body(h1_ref, h2_ref, h3_ref, bt_ref, wa_ref, wb_ref, wc_ref, b_ref,
             pw_ref, pb_ref, out_ref, s_ref, c_ref):
        i = pl.program_id(0)
        hl = (jnp.dot(h1_ref[...], wa_ref[...], preferred_element_type=jnp.float32)
              + jnp.dot(h2_ref[...], wb_ref[...], preferred_element_type=jnp.float32)
              + jnp.dot(h3_ref[...], wc_ref[...], preferred_element_type=jnp.float32)
              + b_ref[...])
        hl = jnp.maximum(hl, 0.0)
        gids = lax.broadcasted_iota(jnp.int32, (G, BLK), 0)
        oh = (bt_ref[0] == gids).astype(jnp.float32)
        ps = jnp.dot(oh, hl, preferred_element_type=jnp.float32)
        cs = jnp.dot(oh, jnp.ones((BLK, H), jnp.float32),
                     preferred_element_type=jnp.float32)

        @pl.when(i == 0)
        def _():
            s_ref[...] = jnp.zeros_like(s_ref)
            c_ref[...] = jnp.zeros_like(c_ref)

        s_ref[...] += ps
        c_ref[...] += cs

        @pl.when(i == nb - 1)
        def _():
            pooled = s_ref[...] / jnp.maximum(c_ref[...], 1.0)
            o = jnp.dot(pooled, pw_ref[...], preferred_element_type=jnp.float32)
            out_ref[...] = jnp.maximum(o + pb_ref[...], 0.0)

    out, _, _ = pl.pallas_call(
        body,
        grid=(nb,),
        in_specs=[_rowspec(), _rowspec(), _rowspec(),
                  pl.BlockSpec((1, 1, BLK), lambda i: (i, 0, 0)),
                  _wspec(), _wspec(), _wspec(), _bspec(), _wspec(), _bspec()],
        out_specs=[pl.BlockSpec((G, H), lambda i: (0, 0)),
                   pl.BlockSpec((G, H), lambda i: (0, 0)),
                   pl.BlockSpec((G, H), lambda i: (0, 0))],
        out_shape=[jax.ShapeDtypeStruct((G, H), jnp.float32),
                   jax.ShapeDtypeStruct((G, H), jnp.float32),
                   jax.ShapeDtypeStruct((G, H), jnp.float32)],
    )(h1, h2, h3, batch2, Wa, Wb, Wc, b, pW, pb)
    return out


def kernel(x, edge_index, batch, params):
    p = params
    src = edge_index[0]
    dst = edge_index[1]
    E = src.shape[0]
    C = -(-E // (NW * EC))
    C = -(-C // 16) * 16
    src_p, dst_p = _tc_pad_edges(src.reshape(E // 128, 128),
                                 dst.reshape(E // 128, 128), C)
    src_w = src_p.reshape(NW, C, EC)
    dst_w = dst_p.reshape(NW, C, EC)
    zrows = jnp.zeros((RPT, H), jnp.float32)
    batch2 = batch.reshape(N // BLK, 1, BLK)

    def r2(a):
        return a.reshape(1, H)

    eps1 = p['eps1'].reshape(1, 1)
    eps2 = p['eps2'].reshape(1, 1)
    eps3 = p['eps3'].reshape(1, 1)

    pr = _sc_agg(x, src_w, dst_w, zrows)
    h1 = _tc_conv1(eps1, x, pr, p['c1_W1'], r2(p['c1_b1']),
                   p['c1_W2'], r2(p['c1_b2']))

    pr = _sc_agg(h1, src_w, dst_w, zrows)
    h2 = _tc_conv_bn(eps2, h1, pr, p['c2_W1'], r2(p['c2_b1']),
                     r2(p['c2_g1']), r2(p['c2_be1']),
                     p['c2_W2'], r2(p['c2_b2']),
                     r2(p['c2_g2']), r2(p['c2_be2']))

    pr = _sc_agg(h2, src_w, dst_w, zrows)
    h3 = _tc_conv_bn(eps3, h2, pr, p['c3_W1'], r2(p['c3_b1']),
                     r2(p['c3_g1']), r2(p['c3_be1']),
                     p['c3_W2'], r2(p['c3_b2']),
                     r2(p['c3_g2']), r2(p['c3_be2']))

    W = p['lin1_W']
    out = _tc_head(h1, h2, h3, batch2,
                   W[0:H], W[H:2 * H], W[2 * H:3 * H], r2(p['lin1_b']),
                   p['pred_W'], r2(p['pred_b']))
    return out
